# baseline (speedup 1.0000x reference)
import functools

import jax
import jax.numpy as jnp
from jax import lax
from jax.experimental import pallas as pl
from jax.experimental.pallas import tpu as pltpu
from jax.experimental.pallas import tpu_sc as plsc

STATE_DIM = 128
ACTION_DIM = 32
IN_DIM = STATE_DIM + ACTION_DIM
HIDDEN = 256
E = 8
OUT_DIM = STATE_DIM + 1
OUT_PAD = 256
X_PAD = 256
SUB = 256

T_R = 512
T_M = 512
NW = 32


def _route_kernel(idx_ref, dest_ref, be_ref):
    blk = idx_ref[:]
    r = lax.broadcasted_iota(jnp.int32, (128, 128), 0)
    c = lax.broadcasted_iota(jnp.int32, (128, 128), 1)
    triu_i = (r <= c).astype(jnp.bfloat16)
    tril_s = (r > c).astype(jnp.bfloat16)

    masks = [(blk == e) for e in range(E)]
    totals = [jnp.sum(m.astype(jnp.float32), keepdims=True).reshape(1, 1)
              for m in masks]
    pcs = [jnp.ceil(t * (1.0 / T_M)) * T_M for t in totals]
    pbs = []
    acc = jnp.zeros((1, 1), jnp.float32)
    for e in range(E):
        pbs.append(acc)
        acc = acc + pcs[e]

    dest = jnp.zeros((128, 128), jnp.float32)
    for e in range(E):
        mb = masks[e].astype(jnp.bfloat16)
        within = jnp.dot(mb, triu_i, preferred_element_type=jnp.float32)
        rowsum = within[:, 127:128].astype(jnp.bfloat16)
        off = jnp.dot(tril_s, rowsum, preferred_element_type=jnp.float32)
        grank = within + off
        dest = jnp.where(masks[e], grank - 1.0 + pbs[e], dest)
    dest_ref[:] = dest.astype(jnp.int32)

    nblk = be_ref.shape[0]
    jv = (lax.broadcasted_iota(jnp.int32, (nblk, 1), 0)
          .astype(jnp.float32) * float(T_M))
    be = jnp.zeros((nblk, 1), jnp.int32)
    for e in range(E):
        be = be + (jv >= (pbs[e] + pcs[e])).astype(jnp.int32)
    be_ref[:] = jnp.minimum(be, E - 1)


def _route(idx_sq, B, NP):
    NB = NP // T_M
    return pl.pallas_call(
        _route_kernel,
        grid=(1,),
        in_specs=[pl.BlockSpec((128, 128), lambda i: (0, 0))],
        out_specs=[
            pl.BlockSpec((128, 128), lambda i: (0, 0)),
            pl.BlockSpec((NB, 1), lambda i: (0, 0)),
        ],
        out_shape=[
            jax.ShapeDtypeStruct((128, 128), jnp.int32),
            jax.ShapeDtypeStruct((NB, 1), jnp.int32),
        ],
        compiler_params=pltpu.CompilerParams(
            dimension_semantics=("arbitrary",)),
    )(idx_sq)


def _make_scatter_x(B, NP, CH):
    mesh = plsc.VectorSubcoreMesh(core_axis_name="c", subcore_axis_name="s")

    nsub = CH // SUB

    @functools.partial(
        pl.kernel, mesh=mesh,
        out_type=jax.ShapeDtypeStruct((NP, X_PAD), jnp.float32),
        scratch_types=[
            pltpu.VMEM((CH // 128, 128), jnp.int32),
            pltpu.VMEM((SUB, X_PAD), jnp.float32),
            pltpu.SemaphoreType.DMA,
        ],
    )
    def scatter_x(state_hbm, action_hbm, dest_hbm, xs_hbm, idx_v, x_v, sem):
        wid = lax.axis_index("s") * 2 + lax.axis_index("c")
        base = wid * CH
        pltpu.sync_copy(dest_hbm.at[wid], idx_v)
        for h in range(nsub):
            bh = base + h * SUB
            pltpu.sync_copy(state_hbm.at[pl.ds(bh, SUB)],
                            x_v.at[:, pl.ds(0, STATE_DIM)])
            pltpu.sync_copy(action_hbm.at[pl.ds(bh, SUB)],
                            x_v.at[:, pl.ds(STATE_DIM, STATE_DIM)])
            copies = [
                pltpu.async_copy(x_v.at[pl.ds(k * 128, 128)],
                                 xs_hbm.at[idx_v.at[h * (SUB // 128) + k]],
                                 sem)
                for k in range(SUB // 128)
            ]
            for c in copies:
                c.wait()

    return scatter_x


def _mlp_kernel(be_ref, x_ref, W1_ref, b1_ref, W2_ref, b2_ref, W3_ref, b3_ref,
                out_ref):
    x = x_ref[:, :IN_DIM].astype(jnp.bfloat16)
    h1 = jnp.maximum(
        jnp.dot(x, W1_ref[0], preferred_element_type=jnp.float32)
        + b1_ref[0], 0.0)
    h2 = jnp.maximum(
        jnp.dot(h1.astype(jnp.bfloat16), W2_ref[0],
                preferred_element_type=jnp.float32) + b2_ref[0], 0.0)
    out_ref[:] = (
        jnp.dot(h2.astype(jnp.bfloat16), W3_ref[0],
                preferred_element_type=jnp.float32) + b3_ref[0])


def _grouped_mlp(be, xs, W1, b1, W2, b2, W3, b3, NP):
    NB = NP // T_M
    grid_spec = pltpu.PrefetchScalarGridSpec(
        num_scalar_prefetch=1,
        grid=(NB,),
        in_specs=[
            pl.BlockSpec((T_M, X_PAD), lambda j, be_r: (j, 0)),
            pl.BlockSpec((1, IN_DIM, HIDDEN), lambda j, be_r: (be_r[j], 0, 0)),
            pl.BlockSpec((1, 1, HIDDEN), lambda j, be_r: (be_r[j], 0, 0)),
            pl.BlockSpec((1, HIDDEN, HIDDEN), lambda j, be_r: (be_r[j], 0, 0)),
            pl.BlockSpec((1, 1, HIDDEN), lambda j, be_r: (be_r[j], 0, 0)),
            pl.BlockSpec((1, HIDDEN, OUT_PAD), lambda j, be_r: (be_r[j], 0, 0)),
            pl.BlockSpec((1, 1, OUT_PAD), lambda j, be_r: (be_r[j], 0, 0)),
        ],
        out_specs=pl.BlockSpec((T_M, OUT_PAD), lambda j, be_r: (j, 0)),
    )
    return pl.pallas_call(
        _mlp_kernel,
        grid_spec=grid_spec,
        out_shape=jax.ShapeDtypeStruct((NP, OUT_PAD), jnp.float32),
        compiler_params=pltpu.CompilerParams(
            dimension_semantics=("arbitrary",)),
    )(be, xs, W1, b1, W2, b2, W3, b3)


def _make_gather_sel(B, NP, CH):
    mesh = plsc.VectorSubcoreMesh(core_axis_name="c", subcore_axis_name="s")

    nsub = CH // SUB

    @functools.partial(
        pl.kernel, mesh=mesh,
        out_type=jax.ShapeDtypeStruct((B, OUT_PAD), jnp.float32),
        scratch_types=[
            pltpu.VMEM((CH // 128, 128), jnp.int32),
            pltpu.VMEM((SUB, OUT_PAD), jnp.float32),
            pltpu.SemaphoreType.DMA,
        ],
    )
    def gather_sel(outs_hbm, dest_hbm, sel_hbm, idx_v, r_v, sem):
        wid = lax.axis_index("s") * 2 + lax.axis_index("c")
        base = wid * CH
        pltpu.sync_copy(dest_hbm.at[wid], idx_v)
        for h in range(nsub):
            copies = [
                pltpu.async_copy(outs_hbm.at[idx_v.at[h * (SUB // 128) + k]],
                                 r_v.at[pl.ds(k * 128, 128)], sem)
                for k in range(SUB // 128)
            ]
            for c in copies:
                c.wait()
            pltpu.sync_copy(r_v, sel_hbm.at[pl.ds(base + h * SUB, SUB)])

    return gather_sel


@jax.jit
def kernel(state, action, W1, b1, W2, b2, W3, b3, idx):
    B = state.shape[0]
    NP = B + E * T_M
    CH = B // NW

    idx_sq = idx.astype(jnp.int32).reshape(128, B // 128)
    dest, be = _route(idx_sq, B, NP)
    dest3 = dest.reshape(NW, CH // 128, 128)

    action_p = jnp.pad(action, ((0, 0), (0, STATE_DIM - ACTION_DIM)))
    xs = _make_scatter_x(B, NP, CH)(state, action_p, dest3)

    W1b = W1.astype(jnp.bfloat16)
    W2b = W2.astype(jnp.bfloat16)
    W3b = jnp.pad(W3, ((0, 0), (0, 0), (0, OUT_PAD - OUT_DIM))).astype(jnp.bfloat16)
    b1r = b1.reshape(E, 1, HIDDEN)
    b2r = b2.reshape(E, 1, HIDDEN)
    b3r = jnp.pad(b3, ((0, 0), (0, OUT_PAD - OUT_DIM))).reshape(E, 1, OUT_PAD)

    outs = _grouped_mlp(be.reshape(NP // T_M), xs,
                        W1b, b1r, W2b, b2r, W3b, b3r, NP)

    sel = _make_gather_sel(B, NP, CH)(outs, dest3)

    next_state = state + sel[:, :STATE_DIM]
    reward = sel[:, STATE_DIM:OUT_DIM]
    STAGE = 4
    if STAGE == 1:
        return (state + dest.astype(jnp.float32), dest.astype(jnp.float32)[:, :1] + be.astype(jnp.float32).sum())
    if STAGE == 2:
        return (state + xs[:B, :STATE_DIM], xs[:B, STATE_DIM:STATE_DIM + 1])
    if STAGE == 3:
        return (state + outs[:B, :STATE_DIM], outs[:B, STATE_DIM:STATE_DIM + 1])
    return (next_state, reward)


# speedup vs baseline: 1.2297x; 1.2297x over previous
import functools

import jax
import jax.numpy as jnp
from jax import lax
from jax.experimental import pallas as pl
from jax.experimental.pallas import tpu as pltpu
from jax.experimental.pallas import tpu_sc as plsc

STATE_DIM = 128
ACTION_DIM = 32
IN_DIM = STATE_DIM + ACTION_DIM
HIDDEN = 256
E = 8
OUT_DIM = STATE_DIM + 1
OUT_PAD = 256
X_PAD = 256
SUB = 256

T_R = 512
T_M = 1024
NW = 32


def _route_kernel(idx_ref, dest_ref, be_ref):
    blk = idx_ref[:]
    r = lax.broadcasted_iota(jnp.int32, (128, 128), 0)
    c = lax.broadcasted_iota(jnp.int32, (128, 128), 1)
    triu_i = (r <= c).astype(jnp.bfloat16)
    tril_s = (r > c).astype(jnp.bfloat16)

    masks = [(blk == e) for e in range(E)]
    totals = [jnp.sum(m.astype(jnp.float32), keepdims=True).reshape(1, 1)
              for m in masks]
    pcs = [jnp.ceil(t * (1.0 / T_M)) * T_M for t in totals]
    pbs = []
    acc = jnp.zeros((1, 1), jnp.float32)
    for e in range(E):
        pbs.append(acc)
        acc = acc + pcs[e]

    dest = jnp.zeros((128, 128), jnp.float32)
    for e in range(E):
        mb = masks[e].astype(jnp.bfloat16)
        within = jnp.dot(mb, triu_i, preferred_element_type=jnp.float32)
        rowsum = within[:, 127:128].astype(jnp.bfloat16)
        off = jnp.dot(tril_s, rowsum, preferred_element_type=jnp.float32)
        grank = within + off
        dest = jnp.where(masks[e], grank - 1.0 + pbs[e], dest)
    dest_ref[:] = dest.astype(jnp.int32)

    nblk = be_ref.shape[0]
    jv = (lax.broadcasted_iota(jnp.int32, (nblk, 1), 0)
          .astype(jnp.float32) * float(T_M))
    be = jnp.zeros((nblk, 1), jnp.int32)
    for e in range(E):
        be = be + (jv >= (pbs[e] + pcs[e])).astype(jnp.int32)
    be_ref[:] = jnp.minimum(be, E - 1)


def _route(idx_sq, B, NP):
    NB = NP // T_M
    return pl.pallas_call(
        _route_kernel,
        grid=(1,),
        in_specs=[pl.BlockSpec((128, 128), lambda i: (0, 0))],
        out_specs=[
            pl.BlockSpec((128, 128), lambda i: (0, 0)),
            pl.BlockSpec((NB, 1), lambda i: (0, 0)),
        ],
        out_shape=[
            jax.ShapeDtypeStruct((128, 128), jnp.int32),
            jax.ShapeDtypeStruct((NB, 1), jnp.int32),
        ],
        compiler_params=pltpu.CompilerParams(
            dimension_semantics=("arbitrary",)),
    )(idx_sq)


def _make_scatter_x(B, NP, CH):
    mesh = plsc.VectorSubcoreMesh(core_axis_name="c", subcore_axis_name="s")
    nsub = CH // 128

    @functools.partial(
        pl.kernel, mesh=mesh,
        out_type=jax.ShapeDtypeStruct((NP, X_PAD), jnp.float32),
        scratch_types=[
            pltpu.VMEM((CH // 128, 128), jnp.int32),
            pltpu.VMEM((128, X_PAD), jnp.float32),
            pltpu.VMEM((128, X_PAD), jnp.float32),
            pltpu.SemaphoreType.DMA,
            pltpu.SemaphoreType.DMA,
            pltpu.SemaphoreType.DMA,
            pltpu.SemaphoreType.DMA,
        ],
    )
    def scatter_x(state_hbm, action_hbm, dest_hbm, xs_hbm,
                  idx_v, x_v0, x_v1, sem0, sem1, isem0, isem1):
        wid = lax.axis_index("s") * 2 + lax.axis_index("c")
        base = wid * CH
        pltpu.sync_copy(dest_hbm.at[wid], idx_v)
        bufs = (x_v0, x_v1)
        sems = (sem0, sem1)
        isems = (isem0, isem1)
        copies = []
        for h in range(nsub):
            if h >= 2:
                copies[h - 2].wait()
            buf = bufs[h % 2]
            bh = base + h * 128
            ca = pltpu.async_copy(state_hbm.at[pl.ds(bh, 128)],
                                  buf.at[:, pl.ds(0, STATE_DIM)],
                                  isems[h % 2])
            cb = pltpu.async_copy(action_hbm.at[pl.ds(bh, 128)],
                                  buf.at[:, pl.ds(STATE_DIM, STATE_DIM)],
                                  isems[h % 2])
            ca.wait()
            cb.wait()
            copies.append(
                pltpu.async_copy(buf, xs_hbm.at[idx_v.at[h]], sems[h % 2]))
        for c in copies[-2:]:
            c.wait()

    return scatter_x


def _mlp_kernel(be_ref, x_ref, W1_ref, b1_ref, W2_ref, b2_ref, W3_ref, b3_ref,
                out_ref):
    x = x_ref[:, :IN_DIM].astype(jnp.bfloat16)
    h1 = jnp.maximum(
        jnp.dot(x, W1_ref[0], preferred_element_type=jnp.float32)
        + b1_ref[0], 0.0)
    h2 = jnp.maximum(
        jnp.dot(h1.astype(jnp.bfloat16), W2_ref[0],
                preferred_element_type=jnp.float32) + b2_ref[0], 0.0)
    out = (jnp.dot(h2.astype(jnp.bfloat16), W3_ref[0],
                   preferred_element_type=jnp.float32) + b3_ref[0])
    out_ref[:, :STATE_DIM] = out[:, :STATE_DIM] + x_ref[:, :STATE_DIM]
    out_ref[:, STATE_DIM:] = out[:, STATE_DIM:]


def _grouped_mlp(be, xs, W1, b1, W2, b2, W3, b3, NP):
    NB = NP // T_M
    grid_spec = pltpu.PrefetchScalarGridSpec(
        num_scalar_prefetch=1,
        grid=(NB,),
        in_specs=[
            pl.BlockSpec((T_M, X_PAD), lambda j, be_r: (j, 0)),
            pl.BlockSpec((1, IN_DIM, HIDDEN), lambda j, be_r: (be_r[j], 0, 0)),
            pl.BlockSpec((1, 1, HIDDEN), lambda j, be_r: (be_r[j], 0, 0)),
            pl.BlockSpec((1, HIDDEN, HIDDEN), lambda j, be_r: (be_r[j], 0, 0)),
            pl.BlockSpec((1, 1, HIDDEN), lambda j, be_r: (be_r[j], 0, 0)),
            pl.BlockSpec((1, HIDDEN, OUT_PAD), lambda j, be_r: (be_r[j], 0, 0)),
            pl.BlockSpec((1, 1, OUT_PAD), lambda j, be_r: (be_r[j], 0, 0)),
        ],
        out_specs=pl.BlockSpec((T_M, OUT_PAD), lambda j, be_r: (j, 0)),
    )
    return pl.pallas_call(
        _mlp_kernel,
        grid_spec=grid_spec,
        out_shape=jax.ShapeDtypeStruct((NP, OUT_PAD), jnp.float32),
        compiler_params=pltpu.CompilerParams(
            dimension_semantics=("arbitrary",)),
    )(be, xs, W1, b1, W2, b2, W3, b3)


def _make_gather_sel(B, NP, CH):
    mesh = plsc.VectorSubcoreMesh(core_axis_name="c", subcore_axis_name="s")
    nsub = CH // SUB

    @functools.partial(
        pl.kernel, mesh=mesh,
        out_type=[
            jax.ShapeDtypeStruct((B, STATE_DIM), jnp.float32),
            jax.ShapeDtypeStruct((B, STATE_DIM), jnp.float32),
        ],
        scratch_types=[
            pltpu.VMEM((CH // 128, 128), jnp.int32),
            pltpu.VMEM((SUB, OUT_PAD), jnp.float32),
            pltpu.SemaphoreType.DMA,
        ],
    )
    def gather_sel(outs_hbm, dest_hbm, ns_hbm, rwb_hbm, idx_v, r_v, sem):
        wid = lax.axis_index("s") * 2 + lax.axis_index("c")
        base = wid * CH
        pltpu.sync_copy(dest_hbm.at[wid], idx_v)
        for h in range(nsub):
            copies = [
                pltpu.async_copy(outs_hbm.at[idx_v.at[h * (SUB // 128) + k]],
                                 r_v.at[pl.ds(k * 128, 128)], sem)
                for k in range(SUB // 128)
            ]
            for c in copies:
                c.wait()
            pltpu.sync_copy(r_v.at[:, pl.ds(0, STATE_DIM)],
                            ns_hbm.at[pl.ds(base + h * SUB, SUB)])
            pltpu.sync_copy(r_v.at[:, pl.ds(STATE_DIM, STATE_DIM)],
                            rwb_hbm.at[pl.ds(base + h * SUB, SUB)])

    return gather_sel


@jax.jit
def kernel(state, action, W1, b1, W2, b2, W3, b3, idx):
    B = state.shape[0]
    NP = B + E * T_M
    CH = B // NW

    idx_sq = idx.astype(jnp.int32).reshape(128, B // 128)
    dest, be = _route(idx_sq, B, NP)
    dest3 = dest.reshape(NW, CH // 128, 128)

    action_p = jnp.pad(action, ((0, 0), (0, STATE_DIM - ACTION_DIM)))
    xs = _make_scatter_x(B, NP, CH)(state, action_p, dest3)

    W1b = W1.astype(jnp.bfloat16)
    W2b = W2.astype(jnp.bfloat16)
    W3b = jnp.pad(W3, ((0, 0), (0, 0), (0, OUT_PAD - OUT_DIM))).astype(jnp.bfloat16)
    b1r = b1.reshape(E, 1, HIDDEN)
    b2r = b2.reshape(E, 1, HIDDEN)
    b3r = jnp.pad(b3, ((0, 0), (0, OUT_PAD - OUT_DIM))).reshape(E, 1, OUT_PAD)

    outs = _grouped_mlp(be.reshape(NP // T_M), xs,
                        W1b, b1r, W2b, b2r, W3b, b3r, NP)

    next_state, rwb = _make_gather_sel(B, NP, CH)(outs, dest3)
    return (next_state, rwb[:, 0:1])
